# baseline (device time: 25907 ns/iter reference)
import jax
import jax.numpy as jnp
from jax import lax
from jax.experimental import pallas as pl
from jax.experimental.pallas import tpu as pltpu

N_DEV = 8
S = 1

J_ORDER = (1, 3, 4, 2, 5, 7, 6)


def kernel(t, W):
    m, k = t.shape
    _, n = W.shape
    E = m // N_DEV
    U = E // S

    def body(t_ref, w_ref, out_ref, tvm_ref, acc_ref, wvm_ref, wbf_ref,
             red_ref, rbuf, ready_sems, tdma_sems,
             rs_send_sems, rs_recv_sems, ag_send_sems, ag_recv_sems):
        my = lax.axis_index("i")
        off_e = my * E

        barrier_sem = pltpu.get_barrier_semaphore()
        pl.semaphore_signal(barrier_sem, inc=1)
        pl.semaphore_wait(barrier_sem, 1)

        tdma = {}
        for j in J_ORDER:
            p = my ^ j
            cp = pltpu.make_async_copy(
                t_ref.at[pl.ds(p * E, E)],
                tvm_ref.at[pl.ds(p * E, E)],
                tdma_sems.at[j - 1],
            )
            cp.start()
            tdma[j] = cp
        own_cp = pltpu.make_async_copy(
            t_ref.at[pl.ds(off_e, E)],
            tvm_ref.at[pl.ds(off_e, E)],
            tdma_sems.at[7],
        )
        own_cp.start()
        w_cp = pltpu.make_async_copy(w_ref, wvm_ref, tdma_sems.at[8])
        w_cp.start()

        for j in range(1, N_DEV):
            pl.semaphore_signal(
                ready_sems.at[j - 1], inc=1,
                device_id=(my ^ j,), device_id_type=pl.DeviceIdType.MESH,
            )

        rs = {}
        for s in range(S):
            for j in J_ORDER:
                p = my ^ j
                if s == 0:
                    pl.semaphore_wait(ready_sems.at[j - 1], 1)
                    tdma[j].wait()
                src = pl.ds(p * E + s * U, U)
                acc_ref[src] = tvm_ref[src].astype(jnp.bfloat16)
                idx = (j - 1) * S + s
                rdma = pltpu.make_async_remote_copy(
                    src_ref=acc_ref.at[src],
                    dst_ref=rbuf.at[idx],
                    send_sem=rs_send_sems.at[idx],
                    recv_sem=rs_recv_sems.at[idx],
                    device_id=(p,),
                    device_id_type=pl.DeviceIdType.MESH,
                )
                rdma.start()
                rs[idx] = rdma

        w_cp.wait()
        wbf_ref[...] = wvm_ref[...].astype(jnp.bfloat16)
        own_cp.wait()

        ag = {}
        for s in range(S):
            sl = pl.ds(off_e + s * U, U)
            red_ref[pl.ds(s * U, U)] = tvm_ref[sl]
            for j in J_ORDER:
                idx = (j - 1) * S + s
                rs[idx].wait()
                red_ref[pl.ds(s * U, U)] += rbuf[idx].astype(jnp.float32)
            acc_ref[sl] = red_ref[pl.ds(s * U, U)].astype(jnp.bfloat16)
            for j in J_ORDER:
                p = my ^ j
                idx = (j - 1) * S + s
                rdma = pltpu.make_async_remote_copy(
                    src_ref=acc_ref.at[sl],
                    dst_ref=acc_ref.at[sl],
                    send_sem=ag_send_sems.at[idx],
                    recv_sem=ag_recv_sems.at[idx],
                    device_id=(p,),
                    device_id_type=pl.DeviceIdType.MESH,
                )
                rdma.start()
                ag[idx] = rdma

        out_ref[pl.ds(off_e, E)] = jax.lax.dot(
            acc_ref[pl.ds(off_e, E)], wbf_ref[...],
            preferred_element_type=jnp.float32,
        )
        for j in J_ORDER:
            p = my ^ j
            for s in range(S):
                ag[(j - 1) * S + s].wait()
            out_ref[pl.ds(p * E, E)] = jax.lax.dot(
                acc_ref[pl.ds(p * E, E)], wbf_ref[...],
                preferred_element_type=jnp.float32,
            )

    return pl.pallas_call(
        body,
        out_shape=jax.ShapeDtypeStruct((m, n), jnp.float32),
        in_specs=[
            pl.BlockSpec(memory_space=pl.ANY),
            pl.BlockSpec(memory_space=pl.ANY),
        ],
        out_specs=pl.BlockSpec(memory_space=pltpu.VMEM),
        scratch_shapes=[
            pltpu.VMEM((m, k), jnp.float32),
            pltpu.VMEM((m, k), jnp.bfloat16),
            pltpu.VMEM((k, n), jnp.float32),
            pltpu.VMEM((k, n), jnp.bfloat16),
            pltpu.VMEM((E, k), jnp.float32),
            pltpu.VMEM(((N_DEV - 1) * S, U, k), jnp.bfloat16),
            pltpu.SemaphoreType.REGULAR((N_DEV - 1,)),
            pltpu.SemaphoreType.DMA((9,)),
            pltpu.SemaphoreType.DMA(((N_DEV - 1) * S,)),
            pltpu.SemaphoreType.DMA(((N_DEV - 1) * S,)),
            pltpu.SemaphoreType.DMA(((N_DEV - 1) * S,)),
            pltpu.SemaphoreType.DMA(((N_DEV - 1) * S,)),
        ],
        compiler_params=pltpu.CompilerParams(collective_id=0),
    )(t, W)


# device time: 25479 ns/iter; 1.0168x vs baseline; 1.0168x over previous
import jax
import jax.numpy as jnp
from jax import lax
from jax.experimental import pallas as pl
from jax.experimental.pallas import tpu as pltpu

N_DEV = 8
S = 1

J_ORDER = (1, 3, 4, 2, 5, 7, 6)


def kernel(t, W):
    m, k = t.shape
    _, n = W.shape
    E = m // N_DEV
    U = E // S

    def body(t_ref, w_ref, out_ref, tvm_ref, acc_ref, wvm_ref, wbf_ref,
             red_ref, rbuf, ready_sems, tdma_sems,
             rs_send_sems, rs_recv_sems, ag_send_sems, ag_recv_sems):
        my = lax.axis_index("i")
        off_e = my * E

        barrier_sem = pltpu.get_barrier_semaphore()
        pl.semaphore_signal(barrier_sem, inc=1)
        pl.semaphore_wait(barrier_sem, 1)

        t_cp = pltpu.make_async_copy(t_ref, tvm_ref, tdma_sems.at[0])
        t_cp.start()
        w_cp = pltpu.make_async_copy(w_ref, wvm_ref, tdma_sems.at[1])
        w_cp.start()

        for j in range(1, N_DEV):
            pl.semaphore_signal(
                ready_sems.at[j - 1], inc=1,
                device_id=(my ^ j,), device_id_type=pl.DeviceIdType.MESH,
            )

        t_cp.wait()
        rs = {}
        for s in range(S):
            for j in J_ORDER:
                p = my ^ j
                if s == 0:
                    pl.semaphore_wait(ready_sems.at[j - 1], 1)
                src = pl.ds(p * E + s * U, U)
                acc_ref[src] = tvm_ref[src].astype(jnp.bfloat16)
                idx = (j - 1) * S + s
                rdma = pltpu.make_async_remote_copy(
                    src_ref=acc_ref.at[src],
                    dst_ref=rbuf.at[idx],
                    send_sem=rs_send_sems.at[idx],
                    recv_sem=rs_recv_sems.at[idx],
                    device_id=(p,),
                    device_id_type=pl.DeviceIdType.MESH,
                )
                rdma.start()
                rs[idx] = rdma

        w_cp.wait()
        wbf_ref[...] = wvm_ref[...].astype(jnp.bfloat16)

        ag = {}
        for s in range(S):
            sl = pl.ds(off_e + s * U, U)
            red_ref[pl.ds(s * U, U)] = tvm_ref[sl]
            for j in J_ORDER:
                idx = (j - 1) * S + s
                rs[idx].wait()
                red_ref[pl.ds(s * U, U)] += rbuf[idx].astype(jnp.float32)
            acc_ref[sl] = red_ref[pl.ds(s * U, U)].astype(jnp.bfloat16)
            for j in J_ORDER:
                p = my ^ j
                idx = (j - 1) * S + s
                rdma = pltpu.make_async_remote_copy(
                    src_ref=acc_ref.at[sl],
                    dst_ref=acc_ref.at[sl],
                    send_sem=ag_send_sems.at[idx],
                    recv_sem=ag_recv_sems.at[idx],
                    device_id=(p,),
                    device_id_type=pl.DeviceIdType.MESH,
                )
                rdma.start()
                ag[idx] = rdma

        out_ref[pl.ds(off_e, E)] = jax.lax.dot(
            acc_ref[pl.ds(off_e, E)], wbf_ref[...],
            preferred_element_type=jnp.float32,
        ).astype(jnp.bfloat16)
        for j in J_ORDER:
            p = my ^ j
            for s in range(S):
                ag[(j - 1) * S + s].wait()
            out_ref[pl.ds(p * E, E)] = jax.lax.dot(
                acc_ref[pl.ds(p * E, E)], wbf_ref[...],
                preferred_element_type=jnp.float32,
            ).astype(jnp.bfloat16)

    return pl.pallas_call(
        body,
        out_shape=jax.ShapeDtypeStruct((m, n), jnp.bfloat16),
        in_specs=[
            pl.BlockSpec(memory_space=pl.ANY),
            pl.BlockSpec(memory_space=pl.ANY),
        ],
        out_specs=pl.BlockSpec(memory_space=pltpu.VMEM),
        scratch_shapes=[
            pltpu.VMEM((m, k), jnp.float32),
            pltpu.VMEM((m, k), jnp.bfloat16),
            pltpu.VMEM((k, n), jnp.float32),
            pltpu.VMEM((k, n), jnp.bfloat16),
            pltpu.VMEM((E, k), jnp.float32),
            pltpu.VMEM(((N_DEV - 1) * S, U, k), jnp.bfloat16),
            pltpu.SemaphoreType.REGULAR((N_DEV - 1,)),
            pltpu.SemaphoreType.DMA((2,)),
            pltpu.SemaphoreType.DMA(((N_DEV - 1) * S,)),
            pltpu.SemaphoreType.DMA(((N_DEV - 1) * S,)),
            pltpu.SemaphoreType.DMA(((N_DEV - 1) * S,)),
            pltpu.SemaphoreType.DMA(((N_DEV - 1) * S,)),
        ],
        compiler_params=pltpu.CompilerParams(collective_id=0),
    )(t, W)


# device time: 22103 ns/iter; 1.1721x vs baseline; 1.1527x over previous
import jax
import jax.numpy as jnp
from jax import lax
from jax.experimental import pallas as pl
from jax.experimental.pallas import tpu as pltpu

N_DEV = 8
S = 2

J_ORDER = (1, 3, 4, 2, 5, 7, 6)


def kernel(t, W):
    m, k = t.shape
    _, n = W.shape
    E = m // N_DEV
    U = E // S

    def body(t_ref, w_ref, out_ref, acc_ref, wbf_ref, red_ref, rbuf,
             ready_sems, rs_send_sems, rs_recv_sems, ag_send_sems,
             ag_recv_sems):
        my = lax.axis_index("i")
        off_e = my * E

        barrier_sem = pltpu.get_barrier_semaphore()
        pl.semaphore_signal(barrier_sem, inc=1)
        pl.semaphore_wait(barrier_sem, 1)

        for j in range(1, N_DEV):
            pl.semaphore_signal(
                ready_sems.at[j - 1], inc=1,
                device_id=(my ^ j,), device_id_type=pl.DeviceIdType.MESH,
            )

        rs = {}
        for s in range(S):
            for j in J_ORDER:
                p = my ^ j
                if s == 0:
                    pl.semaphore_wait(ready_sems.at[j - 1], 1)
                src = pl.ds(p * E + s * U, U)
                acc_ref[src] = t_ref[src].astype(jnp.bfloat16)
                idx = (j - 1) * S + s
                rdma = pltpu.make_async_remote_copy(
                    src_ref=acc_ref.at[src],
                    dst_ref=rbuf.at[idx],
                    send_sem=rs_send_sems.at[idx],
                    recv_sem=rs_recv_sems.at[idx],
                    device_id=(p,),
                    device_id_type=pl.DeviceIdType.MESH,
                )
                rdma.start()
                rs[idx] = rdma

        wbf_ref[...] = w_ref[...].astype(jnp.bfloat16)

        ag = {}
        for s in range(S):
            sl = pl.ds(off_e + s * U, U)
            red_ref[pl.ds(s * U, U)] = t_ref[sl]
            for j in J_ORDER:
                idx = (j - 1) * S + s
                rs[idx].wait()
                red_ref[pl.ds(s * U, U)] += rbuf[idx].astype(jnp.float32)
            acc_ref[sl] = red_ref[pl.ds(s * U, U)].astype(jnp.bfloat16)
            for j in J_ORDER:
                p = my ^ j
                idx = (j - 1) * S + s
                rdma = pltpu.make_async_remote_copy(
                    src_ref=acc_ref.at[sl],
                    dst_ref=acc_ref.at[sl],
                    send_sem=ag_send_sems.at[idx],
                    recv_sem=ag_recv_sems.at[idx],
                    device_id=(p,),
                    device_id_type=pl.DeviceIdType.MESH,
                )
                rdma.start()
                ag[idx] = rdma

        out_ref[pl.ds(off_e, E)] = jax.lax.dot(
            acc_ref[pl.ds(off_e, E)], wbf_ref[...],
            preferred_element_type=jnp.float32,
        ).astype(jnp.bfloat16)
        for j in J_ORDER:
            p = my ^ j
            for s in range(S):
                ag[(j - 1) * S + s].wait()
            out_ref[pl.ds(p * E, E)] = jax.lax.dot(
                acc_ref[pl.ds(p * E, E)], wbf_ref[...],
                preferred_element_type=jnp.float32,
            ).astype(jnp.bfloat16)

    return pl.pallas_call(
        body,
        out_shape=jax.ShapeDtypeStruct((m, n), jnp.bfloat16),
        in_specs=[
            pl.BlockSpec(memory_space=pltpu.VMEM),
            pl.BlockSpec(memory_space=pltpu.VMEM),
        ],
        out_specs=pl.BlockSpec(memory_space=pltpu.VMEM),
        scratch_shapes=[
            pltpu.VMEM((m, k), jnp.bfloat16),
            pltpu.VMEM((k, n), jnp.bfloat16),
            pltpu.VMEM((E, k), jnp.float32),
            pltpu.VMEM(((N_DEV - 1) * S, U, k), jnp.bfloat16),
            pltpu.SemaphoreType.REGULAR((N_DEV - 1,)),
            pltpu.SemaphoreType.DMA(((N_DEV - 1) * S,)),
            pltpu.SemaphoreType.DMA(((N_DEV - 1) * S,)),
            pltpu.SemaphoreType.DMA(((N_DEV - 1) * S,)),
            pltpu.SemaphoreType.DMA(((N_DEV - 1) * S,)),
        ],
        compiler_params=pltpu.CompilerParams(collective_id=0),
    )(t, W)
